# Initial kernel scaffold; baseline (speedup 1.0000x reference)
import jax
import jax.numpy as jnp
from jax.experimental import pallas as pl


def kernel(inputs, table):
    raise NotImplementedError("write your pallas kernel here")


# speedup vs baseline: 3.4704x; 3.4704x over previous
import functools
import math

import jax
import jax.numpy as jnp
from jax import lax
from jax.experimental import pallas as pl
from jax.experimental.pallas import tpu as pltpu
from jax.experimental.pallas import tpu_sc as plsc

_NC = 2
_NS = 16
_NW = _NC * _NS
_LANES = 16

_G = 100
_C = 400
_GC = _C // _G


def _pe_tc_body(out_ref):
    s, d = out_ref.shape
    j = lax.broadcasted_iota(jnp.int32, (s, d), 1)
    pos = lax.broadcasted_iota(jnp.int32, (s, d), 0).astype(jnp.float32) + 1.0
    jeven = (j - (j % 2)).astype(jnp.float32)
    inv_divisor = jnp.exp(jeven * (-math.log(10000.0) / d))
    angle = pos * inv_divisor
    out_ref[...] = jnp.where(j % 2 == 0, jnp.sin(angle), jnp.cos(angle))


def _position_encoding(seq, hidden):
    return pl.pallas_call(
        _pe_tc_body,
        out_shape=jax.ShapeDtypeStruct((seq, hidden), jnp.float32),
    )()


def _sc_body(seq, rows_per_w, idx_hbm, table_hbm, pe_hbm, out_hbm,
             idx_v, dest_v, pe_v, gsem):
    cid = lax.axis_index("c")
    sid = lax.axis_index("s")
    wid = sid * _NC + cid
    n_chunks = rows_per_w // _C
    reps = _C // seq

    pltpu.sync_copy(pe_hbm, pe_v)

    idx_row0 = wid * (rows_per_w // _G)
    row0 = wid * rows_per_w

    def chunk_body(c, carry):
        pltpu.sync_copy(idx_hbm.at[pl.ds(idx_row0 + c * _GC, _GC)], idx_v)
        copies = [
            pltpu.async_copy(table_hbm.at[idx_v.at[j]],
                             dest_v.at[pl.ds(j * _G, _G)], gsem)
            for j in range(_GC)
        ]
        for cp in copies:
            cp.wait()

        def pe_body(s, carry2):
            for k in range(4):
                v = pe_v[s, pl.ds(k * _LANES, _LANES)]
                for r in range(reps):
                    plsc.addupdate(
                        dest_v.at[s + r * seq, pl.ds(k * _LANES, _LANES)], v)
            return carry2

        lax.fori_loop(0, seq, pe_body, 0, unroll=False)

        pltpu.sync_copy(dest_v, out_hbm.at[pl.ds(row0 + c * _C, _C)])
        return carry

    lax.fori_loop(0, n_chunks, chunk_body, 0, unroll=False)


def _sc_gather(idx2d, table, pe, tot_rows):
    seq, hidden = pe.shape
    rows_per_w = tot_rows // _NW
    mesh = plsc.VectorSubcoreMesh(
        core_axis_name="c", subcore_axis_name="s",
        num_cores=_NC, num_subcores=_NS)
    body = functools.partial(_sc_body, seq, rows_per_w)
    return pl.kernel(
        body,
        out_type=jax.ShapeDtypeStruct((tot_rows, hidden), jnp.float32),
        mesh=mesh,
        scratch_types=[
            pltpu.VMEM((_GC, _G), jnp.int32),
            pltpu.VMEM((_C, hidden), jnp.float32),
            pltpu.VMEM((seq, hidden), jnp.float32),
            pltpu.SemaphoreType.DMA,
        ],
        compiler_params=pltpu.CompilerParams(use_tc_tiling_on_sc=False),
    )(idx2d, table, pe)


def kernel(inputs, table):
    batch, seq = inputs.shape
    hidden = table.shape[1]
    tot = batch * seq
    pe = _position_encoding(seq, hidden)
    idx2d = inputs.reshape(tot // _G, _G)
    out = _sc_gather(idx2d, table, pe, tot)
    return out.reshape(batch, seq, hidden)


# speedup vs baseline: 4.2488x; 1.2243x over previous
import functools
import math

import jax
import jax.numpy as jnp
from jax import lax
from jax.experimental import pallas as pl
from jax.experimental.pallas import tpu as pltpu
from jax.experimental.pallas import tpu_sc as plsc

_NC = 2
_NS = 16
_NW = _NC * _NS
_LANES = 16

_G = 100
_C = 200
_GC = _C // _G
_NBUF = 4
_LEAD = 2


def _pe_tc_body(out_ref):
    s, d = out_ref.shape
    j = lax.broadcasted_iota(jnp.int32, (s, d), 1)
    pos = lax.broadcasted_iota(jnp.int32, (s, d), 0).astype(jnp.float32) + 1.0
    jeven = (j - (j % 2)).astype(jnp.float32)
    inv_divisor = jnp.exp(jeven * (-math.log(10000.0) / d))
    angle = pos * inv_divisor
    out_ref[...] = jnp.where(j % 2 == 0, jnp.sin(angle), jnp.cos(angle))


def _position_encoding(seq, hidden):
    return pl.pallas_call(
        _pe_tc_body,
        out_shape=jax.ShapeDtypeStruct((seq, hidden), jnp.float32),
    )()


def _sc_body(seq, rows_per_w, idx_hbm, table_hbm, pe_hbm, out_hbm,
             idx_v, pe_v, dests, gsems, osems):
    cid = lax.axis_index("c")
    sid = lax.axis_index("s")
    wid = sid * _NC + cid
    n_chunks = rows_per_w // _C
    idx_rows = rows_per_w // _G

    pltpu.sync_copy(pe_hbm, pe_v)
    pltpu.sync_copy(idx_hbm.at[pl.ds(wid * idx_rows, idx_rows)], idx_v)
    row0 = wid * rows_per_w

    def gathers(c, b):
        for j in range(_GC):
            pltpu.async_copy(table_hbm.at[idx_v.at[c * _GC + j]],
                             dests[b].at[pl.ds(j * _G, _G)], gsems[b])

    def wait_gathers(c, b):
        for j in range(_GC):
            pltpu.make_async_copy(table_hbm.at[idx_v.at[c * _GC + j]],
                                  dests[b].at[pl.ds(j * _G, _G)],
                                  gsems[b]).wait()

    def scatter(c, b):
        pltpu.async_copy(dests[b], out_hbm.at[pl.ds(row0 + c * _C, _C)],
                         osems[b])

    def wait_scatter(c, b):
        pltpu.make_async_copy(dests[b],
                              out_hbm.at[pl.ds(row0 + c * _C, _C)],
                              osems[b]).wait()

    def pe_add(b):
        dest = dests[b]

        def body(s, carry):
            for k in range(hidden_vregs):
                v = pe_v[s, pl.ds(k * _LANES, _LANES)]
                plsc.addupdate(dest.at[s, pl.ds(k * _LANES, _LANES)], v)
            return carry

        lax.fori_loop(0, seq, body, 0, unroll=4)

    hidden_vregs = pe_v.shape[1] // _LANES

    for b in range(_LEAD):
        gathers(b, b)

    def group(g, carry):
        for b in range(_NBUF):
            c = g * _NBUF + b
            b2 = (b + _LEAD) % _NBUF

            @pl.when(c >= _LEAD)
            def _():
                wait_scatter(c - _LEAD, b2)

            @pl.when(c + _LEAD < n_chunks)
            def _():
                gathers(c + _LEAD, b2)

            wait_gathers(c, b)
            pe_add(b)
            scatter(c, b)
        return carry

    lax.fori_loop(0, n_chunks // _NBUF, group, 0, unroll=False)

    for c in range(n_chunks - _LEAD, n_chunks):
        wait_scatter(c, c % _NBUF)


def _sc_gather(idx2d, table, pe, tot_rows):
    seq, hidden = pe.shape
    rows_per_w = tot_rows // _NW
    mesh = plsc.VectorSubcoreMesh(
        core_axis_name="c", subcore_axis_name="s",
        num_cores=_NC, num_subcores=_NS)
    body = functools.partial(_sc_body, seq, rows_per_w)
    return pl.kernel(
        body,
        out_type=jax.ShapeDtypeStruct((tot_rows, hidden), jnp.float32),
        mesh=mesh,
        scratch_types=[
            pltpu.VMEM((rows_per_w // _G, _G), jnp.int32),
            pltpu.VMEM((seq, hidden), jnp.float32),
            [pltpu.VMEM((_C, hidden), jnp.float32) for _ in range(_NBUF)],
            [pltpu.SemaphoreType.DMA for _ in range(_NBUF)],
            [pltpu.SemaphoreType.DMA for _ in range(_NBUF)],
        ],
        compiler_params=pltpu.CompilerParams(use_tc_tiling_on_sc=False),
    )(idx2d, table, pe)


def kernel(inputs, table):
    batch, seq = inputs.shape
    hidden = table.shape[1]
    tot = batch * seq
    pe = _position_encoding(seq, hidden)
    idx2d = inputs.reshape(tot // _G, _G)
    out = _sc_gather(idx2d, table, pe, tot)
    return out.reshape(batch, seq, hidden)
